# baseline (speedup 1.0000x reference)
import functools

import jax
import jax.numpy as jnp
from jax import lax
from jax.experimental import pallas as pl
from jax.experimental.pallas import tpu as pltpu
from jax.experimental.pallas import tpu_sc as plsc

BATCH = 16384
EMBED = 64
NGENRE = 26


@functools.cache
def _make_sc_gather():
    info = plsc.get_sparse_core_info()
    nc, ns = info.num_cores, info.num_subcores
    nw = nc * ns
    bpw = BATCH // nw
    mesh = plsc.VectorSubcoreMesh(core_axis_name="c", subcore_axis_name="s")

    @functools.partial(
        pl.kernel,
        mesh=mesh,
        out_type=jax.ShapeDtypeStruct((BATCH, EMBED), jnp.float32),
        scratch_types=[
            pltpu.VMEM((bpw,), jnp.int32),
            pltpu.VMEM((bpw, EMBED), jnp.float32),
            pltpu.SemaphoreType.DMA,
        ],
        compiler_params=pltpu.CompilerParams(use_tc_tiling_on_sc=False),
    )
    def sc_gather(table_hbm, idx_hbm, out_hbm, idx_v, rows_v, sem):
        wid = lax.axis_index("s") * nc + lax.axis_index("c")
        base = wid * bpw
        pltpu.sync_copy(idx_hbm.at[pl.ds(base, bpw)], idx_v)
        pltpu.async_copy(table_hbm.at[idx_v], rows_v, sem).wait()
        pltpu.sync_copy(rows_v, out_hbm.at[pl.ds(base, bpw)])

    return sc_gather


def _genre_body(g_ref, wt_ref, out_ref):
    gf = g_ref[...].astype(jnp.float32)
    s = jnp.sum(gf, axis=1, keepdims=True)
    proj = jax.lax.dot_general(
        gf, wt_ref[...], (((1,), (0,)), ((), ())),
        preferred_element_type=jnp.float32)
    out_ref[...] = proj / s


def _genre_tc(g, wt):
    grid = 8
    bs = BATCH // grid
    return pl.pallas_call(
        _genre_body,
        grid=(grid,),
        in_specs=[
            pl.BlockSpec((bs, NGENRE), lambda i: (i, 0)),
            pl.BlockSpec((NGENRE, EMBED), lambda i: (0, 0)),
        ],
        out_specs=pl.BlockSpec((bs, EMBED), lambda i: (i, 0)),
        out_shape=jax.ShapeDtypeStruct((BATCH, EMBED), jnp.float32),
    )(g, wt)


def kernel(year_idx, genre_idx, embedding_year, W_genre):
    idx = year_idx.astype(jnp.int32)
    year_emb = _make_sc_gather()(embedding_year, idx)
    genre_emb = _genre_tc(genre_idx, W_genre.T)
    return jnp.concatenate((year_emb, genre_emb), axis=1)
